# Initial kernel scaffold; baseline (speedup 1.0000x reference)
import jax
import jax.numpy as jnp
from jax.experimental import pallas as pl


def kernel(inputs, Wr, br, W1, b1, W2, b2):
    raise NotImplementedError("write your pallas kernel here")


# speedup vs baseline: 2.1140x; 2.1140x over previous
import functools

import jax
import jax.numpy as jnp
from jax.experimental import pallas as pl
from jax.experimental.pallas import tpu as pltpu

NS = 1
SEQ = 2048
H = 768
MLP = 3072
E = 8
K = 2

M_TILE = 768
M_STEPS = MLP // M_TILE


def _moe_body(x_ref, Wr_ref, br_ref, W1_ref, b1_ref, W2_ref, b2_ref,
              out_ref, aux_ref, comb_ref, xb_ref):
    e = pl.program_id(0)
    m = pl.program_id(1)
    first = jnp.logical_and(e == 0, m == 0)

    @pl.when(first)
    def _router():
        x = x_ref[...]
        logits = jax.lax.dot(x, Wr_ref[...],
                             preferred_element_type=jnp.float32)
        logits = logits + br_ref[...]
        mx = jnp.max(logits, axis=1, keepdims=True)
        ex = jnp.exp(logits - mx)
        probs = ex / jnp.sum(ex, axis=1, keepdims=True)

        lane = jax.lax.broadcasted_iota(jnp.int32, (SEQ, E), 1)
        m1 = jnp.max(probs, axis=1, keepdims=True)
        i1 = jnp.min(jnp.where(probs == m1, lane, E), axis=1, keepdims=True)
        sel1 = lane == i1
        pm = jnp.where(sel1, -jnp.inf, probs)
        m2 = jnp.max(pm, axis=1, keepdims=True)
        i2 = jnp.min(jnp.where(pm == m2, lane, E), axis=1, keepdims=True)
        sel2 = lane == i2
        denom = m1 + m2 + 1e-9
        comb_ref[...] = (jnp.where(sel1, m1, 0.0)
                         + jnp.where(sel2, m2, 0.0)) / denom

        importance = jnp.sum(probs, axis=0)
        load = jnp.sum((probs > 0).astype(jnp.float32), axis=0)
        il = importance * load
        mean = jnp.sum(il) / E
        aux_ref[...] = (jnp.sum((il - mean) ** 2) / E * 0.01).reshape(1, 1)

        xb_ref[...] = x.astype(jnp.bfloat16)
        out_ref[...] = jnp.zeros_like(out_ref)

    xb = xb_ref[...]
    hm = jax.lax.dot(xb, W1_ref[0], preferred_element_type=jnp.float32)
    hm = hm + b1_ref[0]
    hm = jax.nn.gelu(hm, approximate=True)
    y = jax.lax.dot(hm.astype(jnp.bfloat16), W2_ref[0],
                    preferred_element_type=jnp.float32)

    lane = jax.lax.broadcasted_iota(jnp.int32, (SEQ, E), 1)
    col = jnp.sum(jnp.where(lane == e, comb_ref[...], 0.0),
                  axis=1, keepdims=True)

    @pl.when(m == 0)
    def _with_b2():
        out_ref[...] += col * (y + b2_ref[0])

    @pl.when(m != 0)
    def _no_b2():
        out_ref[...] += col * y


def kernel(inputs, Wr, br, W1, b1, W2, b2):
    x = inputs.reshape(SEQ, H)
    W1b = W1.astype(jnp.bfloat16)
    W2b = W2.astype(jnp.bfloat16)

    grid = (E, M_STEPS)
    out, aux = pl.pallas_call(
        _moe_body,
        grid=grid,
        in_specs=[
            pl.BlockSpec((SEQ, H), lambda e, m: (0, 0)),
            pl.BlockSpec((H, E), lambda e, m: (0, 0)),
            pl.BlockSpec((1, E), lambda e, m: (0, 0)),
            pl.BlockSpec((1, H, M_TILE), lambda e, m: (e, 0, m)),
            pl.BlockSpec((1, 1, M_TILE), lambda e, m: (e, 0, m)),
            pl.BlockSpec((1, M_TILE, H), lambda e, m: (e, m, 0)),
            pl.BlockSpec((1, 1, H), lambda e, m: (e, 0, 0)),
        ],
        out_specs=[
            pl.BlockSpec((SEQ, H), lambda e, m: (0, 0)),
            pl.BlockSpec((1, 1), lambda e, m: (0, 0)),
        ],
        out_shape=[
            jax.ShapeDtypeStruct((SEQ, H), jnp.float32),
            jax.ShapeDtypeStruct((1, 1), jnp.float32),
        ],
        scratch_shapes=[
            pltpu.VMEM((SEQ, E), jnp.float32),
            pltpu.VMEM((SEQ, H), jnp.bfloat16),
        ],
    )(x, Wr, br.reshape(1, E), W1b, b1.reshape(E, 1, MLP), W2b,
      b2.reshape(E, 1, H))

    return out.reshape(NS, SEQ, H), aux.reshape(())
